# baseline (speedup 1.0000x reference)
import dataclasses
import functools

import jax
import jax.numpy as jnp
from jax import lax
from jax.experimental import pallas as pl
from jax.experimental.pallas import tpu as pltpu
from jax.experimental.pallas import tpu_sc as plsc

N = 10000
NPAD = 10240
E = 320000
D = 128
DH = 64
K = 10
ALPHA = 0.1

NC = 2
NS = 16
KE = 128
NCHUNK = 162
EPT = KE * NCHUNK
E_ALL = NS * EPT
NODES_PT = NPAD // NS
CHN = 64
NCH_NODE = NODES_PT // CHN
UGRP = 6
NGRP = NCHUNK // UGRP

BM = 2048


def _mlp_body(x_ref, w1_ref, b1_ref, w2_ref, b2_ref, o_ref):
    h = lax.dot_general(x_ref[...], w1_ref[...], (((1,), (0,)), ((), ())),
                        preferred_element_type=jnp.float32,
                        precision=lax.Precision.HIGHEST)
    h = jnp.maximum(h + b1_ref[...], 0.0)
    h = lax.dot_general(h, w2_ref[...], (((1,), (0,)), ((), ())),
                        preferred_element_type=jnp.float32,
                        precision=lax.Precision.HIGHEST)
    h = h + b2_ref[...]
    o_ref[0] = h[:, :DH]
    o_ref[1] = h[:, DH:]


_mlp = pl.pallas_call(
    _mlp_body,
    grid=(NPAD // BM,),
    in_specs=[
        pl.BlockSpec((BM, D), lambda i: (i, 0)),
        pl.BlockSpec((D, D), lambda i: (0, 0)),
        pl.BlockSpec((1, D), lambda i: (0, 0)),
        pl.BlockSpec((D, D), lambda i: (0, 0)),
        pl.BlockSpec((1, D), lambda i: (0, 0)),
    ],
    out_specs=pl.BlockSpec((2, BM, DH), lambda i: (0, i, 0)),
    out_shape=jax.ShapeDtypeStruct((2, NPAD, DH), jnp.float32),
)


def _rsqrt16(d):
    bits = lax.bitcast_convert_type(d, jnp.int32)
    bits = jnp.int32(0x5F3759DF) - lax.shift_right_arithmetic(bits, 1)
    y = lax.bitcast_convert_type(bits, jnp.float32)
    y = y * (1.5 - 0.5 * d * y * y)
    y = y * (1.5 - 0.5 * d * y * y)
    y = y * (1.5 - 0.5 * d * y * y)
    return y


_sc_params = pltpu.CompilerParams()
if "needs_layout_passes" in pltpu.CompilerParams.__dataclass_fields__:
    _sc_params = dataclasses.replace(_sc_params, needs_layout_passes=False)
if "use_tc_tiling_on_sc" in pltpu.CompilerParams.__dataclass_fields__:
    _sc_params = dataclasses.replace(_sc_params, use_tc_tiling_on_sc=False)


@functools.partial(
    pl.kernel,
    out_type=(
        jax.ShapeDtypeStruct((NC, NPAD, DH), jnp.float32),
        jax.ShapeDtypeStruct((NS, NCHUNK, 3, KE), jnp.int32),
    ),
    mesh=plsc.VectorSubcoreMesh(core_axis_name="c", subcore_axis_name="s"),
    compiler_params=_sc_params,
    scratch_types=[
        pltpu.VMEM((NPAD,), jnp.float32),
        pltpu.VMEM((KE, DH), jnp.float32),
        pltpu.VMEM((KE, DH), jnp.float32),
        pltpu.VMEM((3, KE), jnp.int32),
        pltpu.VMEM((3, KE), jnp.int32),
        pltpu.VMEM((3, KE), jnp.int32),
        pltpu.VMEM((CHN, DH), jnp.float32),
        pltpu.VMEM((CHN, DH), jnp.float32),
        pltpu.VMEM((CHN, DH), jnp.float32),
        pltpu.VMEM((NODES_PT,), jnp.float32),
        pltpu.VMEM((KE,), jnp.float32),
        pltpu.SemaphoreType.DMA,
        pltpu.SemaphoreType.DMA,
        pltpu.SemaphoreType.DMA,
        pltpu.SemaphoreType.DMA,
        pltpu.SemaphoreType.DMA,
        pltpu.SemaphoreType.DMA,
        pltpu.SemaphoreType.DMA,
        pltpu.VMEM_SHARED((NPAD, DH), jnp.float32),
        pltpu.VMEM_SHARED((NPAD, DH), jnp.float32),
        pltpu.VMEM_SHARED((NPAD,), jnp.float32),
    ],
)
def _appnp_sc(h_hbm, idxw_hbm, out_hbm, pk_hbm,
              dinv_t, rows0, rows1, icn0, icn1, icn2, na, nh, zb, deg_t, cnb,
              sem_i0, sem_i1, sem_i2, sem_g0, sem_g1, sem_s0, sem_s1,
              out_sh, agg_sh, deg_sh):
    c = lax.axis_index("c")
    s = lax.axis_index("s")
    base = s * NODES_PT
    zvec = jnp.zeros((16,), jnp.float32)
    rows = [rows0, rows1]
    icn = [icn0, icn1, icn2]
    sem_i = [sem_i0, sem_i1, sem_i2]
    sem_g = [sem_g0, sem_g1]
    sem_s = [sem_s0, sem_s1]

    @pl.loop(0, CHN)
    def _(i):
        for j in range(DH // 16):
            zb[i, pl.ds(j * 16, 16)] = zvec

    @pl.loop(0, NODES_PT // 16)
    def _(i):
        deg_t[pl.ds(i * 16, 16)] = zvec

    pltpu.sync_copy(deg_t, deg_sh.at[pl.ds(base, NODES_PT)])
    plsc.subcore_barrier()

    @pl.loop(0, NCHUNK)
    def _(cc):
        pltpu.sync_copy(idxw_hbm.at[s, cc], icn0)

        @pl.loop(0, KE // 16)
        def _(i):
            slc = pl.ds(i * 16, 16)
            cnb[slc] = plsc.bitcast(icn0[2, slc], jnp.float32)

        pltpu.sync_copy(cnb, deg_sh.at[icn0.at[1]], add=True)

    plsc.subcore_barrier()

    pltpu.sync_copy(deg_sh.at[pl.ds(base, NODES_PT)], deg_t)

    @pl.loop(0, NODES_PT // 16)
    def _(i):
        deg_t[pl.ds(i * 16, 16)] = _rsqrt16(deg_t[pl.ds(i * 16, 16)])

    pltpu.sync_copy(deg_t, deg_sh.at[pl.ds(base, NODES_PT)])
    plsc.subcore_barrier()

    pltpu.sync_copy(deg_sh, dinv_t)

    @pl.loop(0, NCHUNK)
    def _(cc):
        pltpu.sync_copy(idxw_hbm.at[s, cc], icn1)

        @pl.loop(0, KE // 16)
        def _(i):
            slc = pl.ds(i * 16, 16)
            dr = plsc.load_gather(dinv_t, [icn1[0, slc]])
            dc = plsc.load_gather(dinv_t, [icn1[1, slc]])
            w = plsc.bitcast(icn1[2, slc], jnp.float32)
            icn1[2, slc] = plsc.bitcast(w * dr * dc, jnp.int32)

        pltpu.sync_copy(icn1, pk_hbm.at[s, cc])

    @pl.loop(0, NCH_NODE)
    def _(j):
        nbase = base + j * CHN
        pltpu.sync_copy(h_hbm.at[c, pl.ds(nbase, CHN)], na)
        pltpu.sync_copy(na, out_sh.at[pl.ds(nbase, CHN)])
        pltpu.sync_copy(zb, agg_sh.at[pl.ds(nbase, CHN)])

    plsc.subcore_barrier()

    def scale_rows(rb, ic):
        @pl.loop(0, KE // 16)
        def _(i16):
            cn = plsc.bitcast(ic[2, pl.ds(i16 * 16, 16)], jnp.float32)
            for ii in range(16):
                sc = cn[ii]
                for j in range(DH // 16):
                    slc = pl.ds(j * 16, 16)
                    rb[i16 * 16 + ii, slc] = rb[i16 * 16 + ii, slc] * sc

    @pl.loop(0, K)
    def _(it):
        pltpu.sync_copy(pk_hbm.at[s, 0], icn0)
        pltpu.async_copy(pk_hbm.at[s, 1], icn1, sem_i1)
        @pl.loop(0, KE)
        def _(i):
            for j in range(DH // 16):
                rows1[i, pl.ds(j * 16, 16)] = zvec

        pltpu.async_copy(rows1, agg_sh.at[icn0.at[1]], sem_s1, add=True)
        pltpu.async_copy(out_sh.at[icn0.at[0]], rows0, sem_g0)

        @pl.loop(0, NGRP)
        def _(g):
            cbase = g * UGRP
            for u in range(UGRP):
                cc = cbase + u
                s2, s2n = u % 2, (u + 1) % 2
                s3, s3n, s3p = u % 3, (u + 1) % 3, (u + 2) % 3
                cn1 = lax.rem(cc + 1, NCHUNK)
                cn2 = lax.rem(cc + 2, NCHUNK)
                pltpu.make_async_copy(
                    pk_hbm.at[s, cn1], icn[s3n], sem_i[s3n]).wait()
                pltpu.make_async_copy(
                    out_sh.at[icn[s3].at[0]], rows[s2], sem_g[s2]).wait()
                pltpu.make_async_copy(
                    rows[s2n], agg_sh.at[icn[s3p].at[1]], sem_s[s2n]).wait()
                pltpu.async_copy(pk_hbm.at[s, cn2], icn[s3p], sem_i[s3p])
                pltpu.async_copy(
                    out_sh.at[icn[s3n].at[0]], rows[s2n], sem_g[s2n])
                scale_rows(rows[s2], icn[s3])
                pltpu.async_copy(
                    rows[s2], agg_sh.at[icn[s3].at[1]], sem_s[s2], add=True)

        pltpu.make_async_copy(
            rows[1], agg_sh.at[icn[2].at[1]], sem_s[1]).wait()
        pltpu.make_async_copy(out_sh.at[icn[0].at[0]], rows[0], sem_g[0]).wait()
        pltpu.make_async_copy(pk_hbm.at[s, 1], icn[1], sem_i[1]).wait()
        plsc.subcore_barrier()

        @pl.loop(0, NCH_NODE)
        def _(j):
            nbase = base + j * CHN
            pltpu.sync_copy(agg_sh.at[pl.ds(nbase, CHN)], na)
            pltpu.sync_copy(h_hbm.at[c, pl.ds(nbase, CHN)], nh)
            pltpu.sync_copy(zb, agg_sh.at[pl.ds(nbase, CHN)])

            @pl.loop(0, CHN)
            def _(i):
                for jj in range(DH // 16):
                    slc = pl.ds(jj * 16, 16)
                    na[i, slc] = (1.0 - ALPHA) * na[i, slc] + ALPHA * nh[i, slc]

            pltpu.sync_copy(na, out_sh.at[pl.ds(nbase, CHN)])

        plsc.subcore_barrier()

    @pl.loop(0, NCH_NODE)
    def _(j):
        nbase = base + j * CHN
        pltpu.sync_copy(out_sh.at[pl.ds(nbase, CHN)], na)
        pltpu.sync_copy(na, out_hbm.at[c, pl.ds(nbase, CHN)])


def kernel(x, edge_index, edge_weight, W1, b1, W2, b2):
    x_pad = jnp.pad(x, ((0, NPAD - N), (0, 0)))
    h2 = _mlp(x_pad, W1, b1.reshape(1, D), W2, b2.reshape(1, D))

    npad_e = E_ALL - E - NPAD
    loop_idx = jnp.arange(NPAD, dtype=jnp.int32)
    zpad = jnp.zeros((npad_e,), jnp.int32)
    rows_all = jnp.concatenate([edge_index[0], loop_idx, zpad])
    cols_all = jnp.concatenate([edge_index[1], loop_idx, zpad])
    w_all = jnp.concatenate(
        [edge_weight, jnp.ones((NPAD,), jnp.float32),
         jnp.zeros((npad_e,), jnp.float32)])
    wbits = lax.bitcast_convert_type(w_all, jnp.int32)
    idxw = jnp.stack([rows_all.reshape(NS, NCHUNK, KE),
                      cols_all.reshape(NS, NCHUNK, KE),
                      wbits.reshape(NS, NCHUNK, KE)], axis=2)

    out2, _ = _appnp_sc(h2, idxw)
    return jnp.concatenate([out2[0], out2[1]], axis=1)[:N]


# speedup vs baseline: 1.0288x; 1.0288x over previous
import dataclasses
import functools

import jax
import jax.numpy as jnp
from jax import lax
from jax.experimental import pallas as pl
from jax.experimental.pallas import tpu as pltpu
from jax.experimental.pallas import tpu_sc as plsc

N = 10000
NPAD = 10240
E = 320000
D = 128
DH = 64
K = 10
ALPHA = 0.1

NC = 2
NS = 16
KE = 128
NCHUNK = 162
EPT = KE * NCHUNK
E_ALL = NS * EPT
NODES_PT = NPAD // NS
CHN = 64
NCH_NODE = NODES_PT // CHN
UGRP = 6
NGRP = NCHUNK // UGRP

BM = 2048


def _mlp_body(x_ref, w1_ref, b1_ref, w2_ref, b2_ref, o_ref):
    h = lax.dot_general(x_ref[...], w1_ref[...], (((1,), (0,)), ((), ())),
                        preferred_element_type=jnp.float32,
                        precision=lax.Precision.HIGHEST)
    h = jnp.maximum(h + b1_ref[...], 0.0)
    h = lax.dot_general(h, w2_ref[...], (((1,), (0,)), ((), ())),
                        preferred_element_type=jnp.float32,
                        precision=lax.Precision.HIGHEST)
    h = h + b2_ref[...]
    o_ref[0] = h[:, :DH]
    o_ref[1] = h[:, DH:]


_mlp = pl.pallas_call(
    _mlp_body,
    grid=(NPAD // BM,),
    in_specs=[
        pl.BlockSpec((BM, D), lambda i: (i, 0)),
        pl.BlockSpec((D, D), lambda i: (0, 0)),
        pl.BlockSpec((1, D), lambda i: (0, 0)),
        pl.BlockSpec((D, D), lambda i: (0, 0)),
        pl.BlockSpec((1, D), lambda i: (0, 0)),
    ],
    out_specs=pl.BlockSpec((2, BM, DH), lambda i: (0, i, 0)),
    out_shape=jax.ShapeDtypeStruct((2, NPAD, DH), jnp.float32),
)


def _rsqrt16(d):
    bits = lax.bitcast_convert_type(d, jnp.int32)
    bits = jnp.int32(0x5F3759DF) - lax.shift_right_arithmetic(bits, 1)
    y = lax.bitcast_convert_type(bits, jnp.float32)
    y = y * (1.5 - 0.5 * d * y * y)
    y = y * (1.5 - 0.5 * d * y * y)
    y = y * (1.5 - 0.5 * d * y * y)
    return y


_sc_params = pltpu.CompilerParams()
if "needs_layout_passes" in pltpu.CompilerParams.__dataclass_fields__:
    _sc_params = dataclasses.replace(_sc_params, needs_layout_passes=False)
if "use_tc_tiling_on_sc" in pltpu.CompilerParams.__dataclass_fields__:
    _sc_params = dataclasses.replace(_sc_params, use_tc_tiling_on_sc=False)


@functools.partial(
    pl.kernel,
    out_type=(
        jax.ShapeDtypeStruct((NC, NPAD, DH), jnp.float32),
        jax.ShapeDtypeStruct((NS, NCHUNK, 3, KE), jnp.int32),
    ),
    mesh=plsc.VectorSubcoreMesh(core_axis_name="c", subcore_axis_name="s"),
    compiler_params=_sc_params,
    scratch_types=[
        pltpu.VMEM((NPAD,), jnp.float32),
        pltpu.VMEM((KE, DH), jnp.float32),
        pltpu.VMEM((KE, DH), jnp.float32),
        pltpu.VMEM((3, KE), jnp.int32),
        pltpu.VMEM((3, KE), jnp.int32),
        pltpu.VMEM((3, KE), jnp.int32),
        pltpu.VMEM((CHN, DH), jnp.float32),
        pltpu.VMEM((CHN, DH), jnp.float32),
        pltpu.VMEM((CHN, DH), jnp.float32),
        pltpu.VMEM((NODES_PT,), jnp.float32),
        pltpu.VMEM((KE,), jnp.float32),
        pltpu.SemaphoreType.DMA,
        pltpu.SemaphoreType.DMA,
        pltpu.SemaphoreType.DMA,
        pltpu.SemaphoreType.DMA,
        pltpu.SemaphoreType.DMA,
        pltpu.SemaphoreType.DMA,
        pltpu.SemaphoreType.DMA,
        pltpu.VMEM_SHARED((NPAD, DH), jnp.float32),
        pltpu.VMEM_SHARED((NPAD, DH), jnp.float32),
        pltpu.VMEM_SHARED((NPAD,), jnp.float32),
    ],
)
def _appnp_sc(h_hbm, idxw_hbm, out_hbm, pk_hbm,
              dinv_t, rows0, rows1, icn0, icn1, icn2, na, nh, zb, deg_t, cnb,
              sem_i0, sem_i1, sem_i2, sem_g0, sem_g1, sem_s0, sem_s1,
              out_sh, agg_sh, deg_sh):
    c = lax.axis_index("c")
    s = lax.axis_index("s")
    base = s * NODES_PT
    zvec = jnp.zeros((16,), jnp.float32)
    rows = [rows0, rows1]
    icn = [icn0, icn1, icn2]
    sem_i = [sem_i0, sem_i1, sem_i2]
    sem_g = [sem_g0, sem_g1]
    sem_s = [sem_s0, sem_s1]

    @pl.loop(0, CHN)
    def _(i):
        for j in range(DH // 16):
            zb[i, pl.ds(j * 16, 16)] = zvec

    @pl.loop(0, NODES_PT // 16)
    def _(i):
        deg_t[pl.ds(i * 16, 16)] = zvec

    pltpu.sync_copy(deg_t, deg_sh.at[pl.ds(base, NODES_PT)])
    plsc.subcore_barrier()

    pltpu.async_copy(idxw_hbm.at[s, 0], icn0, sem_i0)
    pltpu.async_copy(idxw_hbm.at[s, 1], icn1, sem_i1)

    @pl.loop(0, NCHUNK // 2)
    def _(g):
        for u in range(2):
            cc = g * 2 + u
            buf, sem = icn[u], sem_i[u]
            pltpu.make_async_copy(idxw_hbm.at[s, cc], buf, sem).wait()

            @pl.loop(0, KE // 16)
            def _(i):
                slc = pl.ds(i * 16, 16)
                cnb[slc] = plsc.bitcast(buf[2, slc], jnp.float32)

            pltpu.sync_copy(cnb, deg_sh.at[buf.at[1]], add=True)
            pltpu.async_copy(
                idxw_hbm.at[s, lax.rem(cc + 2, NCHUNK)], buf, sem)

    pltpu.make_async_copy(idxw_hbm.at[s, 0], icn0, sem_i0).wait()
    pltpu.make_async_copy(idxw_hbm.at[s, 1], icn1, sem_i1).wait()

    plsc.subcore_barrier()

    pltpu.sync_copy(deg_sh.at[pl.ds(base, NODES_PT)], deg_t)

    @pl.loop(0, NODES_PT // 16)
    def _(i):
        deg_t[pl.ds(i * 16, 16)] = _rsqrt16(deg_t[pl.ds(i * 16, 16)])

    pltpu.sync_copy(deg_t, deg_sh.at[pl.ds(base, NODES_PT)])
    plsc.subcore_barrier()

    pltpu.sync_copy(deg_sh, dinv_t)

    pltpu.async_copy(idxw_hbm.at[s, 0], icn0, sem_i0)
    pltpu.async_copy(idxw_hbm.at[s, 1], icn1, sem_i1)

    @pl.loop(0, NCHUNK // 2)
    def _(g):
        for u in range(2):
            cc = g * 2 + u
            buf, sem = icn[u], sem_i[u]
            pltpu.make_async_copy(idxw_hbm.at[s, cc], buf, sem).wait()

            @pl.loop(0, KE // 16)
            def _(i):
                slc = pl.ds(i * 16, 16)
                dr = plsc.load_gather(dinv_t, [buf[0, slc]])
                dc = plsc.load_gather(dinv_t, [buf[1, slc]])
                w = plsc.bitcast(buf[2, slc], jnp.float32)
                buf[2, slc] = plsc.bitcast(w * dr * dc, jnp.int32)

            pltpu.sync_copy(buf, pk_hbm.at[s, cc])
            pltpu.async_copy(
                idxw_hbm.at[s, lax.rem(cc + 2, NCHUNK)], buf, sem)

    pltpu.make_async_copy(idxw_hbm.at[s, 0], icn0, sem_i0).wait()
    pltpu.make_async_copy(idxw_hbm.at[s, 1], icn1, sem_i1).wait()

    @pl.loop(0, NCH_NODE)
    def _(j):
        nbase = base + j * CHN
        pltpu.sync_copy(h_hbm.at[c, pl.ds(nbase, CHN)], na)
        pltpu.sync_copy(na, out_sh.at[pl.ds(nbase, CHN)])
        pltpu.sync_copy(zb, agg_sh.at[pl.ds(nbase, CHN)])

    plsc.subcore_barrier()

    def scale_rows(rb, ic):
        @pl.loop(0, KE // 16)
        def _(i16):
            cn = plsc.bitcast(ic[2, pl.ds(i16 * 16, 16)], jnp.float32)
            for ii in range(16):
                sc = cn[ii]
                for j in range(DH // 16):
                    slc = pl.ds(j * 16, 16)
                    rb[i16 * 16 + ii, slc] = rb[i16 * 16 + ii, slc] * sc

    @pl.loop(0, K)
    def _(it):
        pltpu.sync_copy(pk_hbm.at[s, 0], icn0)
        pltpu.async_copy(pk_hbm.at[s, 1], icn1, sem_i1)
        @pl.loop(0, KE)
        def _(i):
            for j in range(DH // 16):
                rows1[i, pl.ds(j * 16, 16)] = zvec

        pltpu.async_copy(rows1, agg_sh.at[icn0.at[1]], sem_s1, add=True)
        pltpu.async_copy(out_sh.at[icn0.at[0]], rows0, sem_g0)

        @pl.loop(0, NGRP)
        def _(g):
            cbase = g * UGRP
            for u in range(UGRP):
                cc = cbase + u
                s2, s2n = u % 2, (u + 1) % 2
                s3, s3n, s3p = u % 3, (u + 1) % 3, (u + 2) % 3
                cn1 = lax.rem(cc + 1, NCHUNK)
                cn2 = lax.rem(cc + 2, NCHUNK)
                pltpu.make_async_copy(
                    pk_hbm.at[s, cn1], icn[s3n], sem_i[s3n]).wait()
                pltpu.make_async_copy(
                    out_sh.at[icn[s3].at[0]], rows[s2], sem_g[s2]).wait()
                pltpu.make_async_copy(
                    rows[s2n], agg_sh.at[icn[s3p].at[1]], sem_s[s2n]).wait()
                pltpu.async_copy(pk_hbm.at[s, cn2], icn[s3p], sem_i[s3p])
                pltpu.async_copy(
                    out_sh.at[icn[s3n].at[0]], rows[s2n], sem_g[s2n])
                scale_rows(rows[s2], icn[s3])
                pltpu.async_copy(
                    rows[s2], agg_sh.at[icn[s3].at[1]], sem_s[s2], add=True)

        pltpu.make_async_copy(
            rows[1], agg_sh.at[icn[2].at[1]], sem_s[1]).wait()
        pltpu.make_async_copy(out_sh.at[icn[0].at[0]], rows[0], sem_g[0]).wait()
        pltpu.make_async_copy(pk_hbm.at[s, 1], icn[1], sem_i[1]).wait()
        plsc.subcore_barrier()

        @pl.loop(0, NCH_NODE)
        def _(j):
            nbase = base + j * CHN
            pltpu.async_copy(agg_sh.at[pl.ds(nbase, CHN)], na, sem_g0)
            pltpu.async_copy(h_hbm.at[c, pl.ds(nbase, CHN)], nh, sem_g1)
            pltpu.make_async_copy(
                agg_sh.at[pl.ds(nbase, CHN)], na, sem_g0).wait()
            pltpu.make_async_copy(
                h_hbm.at[c, pl.ds(nbase, CHN)], nh, sem_g1).wait()
            pltpu.sync_copy(zb, agg_sh.at[pl.ds(nbase, CHN)])

            @pl.loop(0, CHN)
            def _(i):
                for jj in range(DH // 16):
                    slc = pl.ds(jj * 16, 16)
                    na[i, slc] = (1.0 - ALPHA) * na[i, slc] + ALPHA * nh[i, slc]

            pltpu.sync_copy(na, out_sh.at[pl.ds(nbase, CHN)])

        plsc.subcore_barrier()

    @pl.loop(0, NCH_NODE)
    def _(j):
        nbase = base + j * CHN
        pltpu.sync_copy(out_sh.at[pl.ds(nbase, CHN)], na)
        pltpu.sync_copy(na, out_hbm.at[c, pl.ds(nbase, CHN)])


def kernel(x, edge_index, edge_weight, W1, b1, W2, b2):
    x_pad = jnp.pad(x, ((0, NPAD - N), (0, 0)))
    h2 = _mlp(x_pad, W1, b1.reshape(1, D), W2, b2.reshape(1, D))

    npad_e = E_ALL - E - NPAD
    loop_idx = jnp.arange(NPAD, dtype=jnp.int32)
    zpad = jnp.zeros((npad_e,), jnp.int32)
    rows_all = jnp.concatenate([edge_index[0], loop_idx, zpad])
    cols_all = jnp.concatenate([edge_index[1], loop_idx, zpad])
    w_all = jnp.concatenate(
        [edge_weight, jnp.ones((NPAD,), jnp.float32),
         jnp.zeros((npad_e,), jnp.float32)])
    wbits = lax.bitcast_convert_type(w_all, jnp.int32)
    idxw = jnp.stack([rows_all.reshape(NS, NCHUNK, KE),
                      cols_all.reshape(NS, NCHUNK, KE),
                      wbits.reshape(NS, NCHUNK, KE)], axis=2)

    out2, _ = _appnp_sc(h2, idxw)
    return jnp.concatenate([out2[0], out2[1]], axis=1)[:N]


# speedup vs baseline: 1.7661x; 1.7166x over previous
import dataclasses
import functools

import jax
import jax.numpy as jnp
from jax import lax
from jax.experimental import pallas as pl
from jax.experimental.pallas import tpu as pltpu
from jax.experimental.pallas import tpu_sc as plsc

N = 10000
NPAD = 10240
E = 320000
D = 128
DH = 64
K = 10
ALPHA = 0.1

NC = 2
NS = 16
KE = 128
NCHUNK = 162
EPT = KE * NCHUNK
E_ALL = NS * EPT
NODES_PT = NPAD // NS
CHN = 64
NCH_NODE = NODES_PT // CHN
UGRP = 6
NGRP = NCHUNK // UGRP

BM = 2048


def _mlp_body(x_ref, w1_ref, b1_ref, w2_ref, b2_ref, o_ref):
    h = lax.dot_general(x_ref[...], w1_ref[...], (((1,), (0,)), ((), ())),
                        preferred_element_type=jnp.float32,
                        precision=lax.Precision.HIGHEST)
    h = jnp.maximum(h + b1_ref[...], 0.0)
    h = lax.dot_general(h, w2_ref[...], (((1,), (0,)), ((), ())),
                        preferred_element_type=jnp.float32,
                        precision=lax.Precision.HIGHEST)
    h = h + b2_ref[...]
    o_ref[0] = h[:, :DH]
    o_ref[1] = h[:, DH:]


_mlp = pl.pallas_call(
    _mlp_body,
    grid=(NPAD // BM,),
    in_specs=[
        pl.BlockSpec((BM, D), lambda i: (i, 0)),
        pl.BlockSpec((D, D), lambda i: (0, 0)),
        pl.BlockSpec((1, D), lambda i: (0, 0)),
        pl.BlockSpec((D, D), lambda i: (0, 0)),
        pl.BlockSpec((1, D), lambda i: (0, 0)),
    ],
    out_specs=pl.BlockSpec((2, BM, DH), lambda i: (0, i, 0)),
    out_shape=jax.ShapeDtypeStruct((2, NPAD, DH), jnp.float32),
)


def _rsqrt16(d):
    bits = lax.bitcast_convert_type(d, jnp.int32)
    bits = jnp.int32(0x5F3759DF) - lax.shift_right_arithmetic(bits, 1)
    y = lax.bitcast_convert_type(bits, jnp.float32)
    y = y * (1.5 - 0.5 * d * y * y)
    y = y * (1.5 - 0.5 * d * y * y)
    y = y * (1.5 - 0.5 * d * y * y)
    return y


_sc_params = pltpu.CompilerParams()
if "needs_layout_passes" in pltpu.CompilerParams.__dataclass_fields__:
    _sc_params = dataclasses.replace(_sc_params, needs_layout_passes=False)
if "use_tc_tiling_on_sc" in pltpu.CompilerParams.__dataclass_fields__:
    _sc_params = dataclasses.replace(_sc_params, use_tc_tiling_on_sc=False)


@functools.partial(
    pl.kernel,
    out_type=(
        jax.ShapeDtypeStruct((NC, NPAD, DH), jnp.float32),
        jax.ShapeDtypeStruct((NS, NCHUNK, 3, KE), jnp.int32),
    ),
    mesh=plsc.VectorSubcoreMesh(core_axis_name="c", subcore_axis_name="s"),
    compiler_params=_sc_params,
    scratch_types=[
        pltpu.VMEM((NPAD,), jnp.float32),
        pltpu.VMEM((KE, DH), jnp.float32),
        pltpu.VMEM((KE, DH), jnp.float32),
        pltpu.VMEM((3, KE), jnp.int32),
        pltpu.VMEM((3, KE), jnp.int32),
        pltpu.VMEM((3, KE), jnp.int32),
        pltpu.VMEM((CHN, DH), jnp.float32),
        pltpu.VMEM((CHN, DH), jnp.float32),
        pltpu.VMEM((CHN, DH), jnp.float32),
        pltpu.VMEM((NODES_PT,), jnp.float32),
        pltpu.VMEM((KE,), jnp.float32),
        pltpu.SemaphoreType.DMA,
        pltpu.SemaphoreType.DMA,
        pltpu.SemaphoreType.DMA,
        pltpu.SemaphoreType.DMA,
        pltpu.SemaphoreType.DMA,
        pltpu.SemaphoreType.DMA,
        pltpu.SemaphoreType.DMA,
        pltpu.VMEM_SHARED((NPAD, DH), jnp.float32),
        pltpu.VMEM_SHARED((NPAD, DH), jnp.float32),
        pltpu.VMEM_SHARED((NPAD,), jnp.float32),
    ],
)
def _appnp_sc(h_hbm, idxw_hbm, out_hbm, pk_hbm,
              dinv_t, rows0, rows1, icn0, icn1, icn2, na, nh, zb, deg_t, cnb,
              sem_i0, sem_i1, sem_i2, sem_g0, sem_g1, sem_s0, sem_s1,
              out_sh, agg_sh, deg_sh):
    c = lax.axis_index("c")
    s = lax.axis_index("s")
    base = s * NODES_PT
    zvec = jnp.zeros((16,), jnp.float32)
    rows = [rows0, rows1]
    icn = [icn0, icn1, icn2]
    sem_i = [sem_i0, sem_i1, sem_i2]
    sem_g = [sem_g0, sem_g1]
    sem_s = [sem_s0, sem_s1]

    @pl.loop(0, CHN)
    def _(i):
        for j in range(DH // 16):
            zb[i, pl.ds(j * 16, 16)] = zvec

    @pl.loop(0, NODES_PT // 16)
    def _(i):
        deg_t[pl.ds(i * 16, 16)] = zvec

    pltpu.sync_copy(deg_t, deg_sh.at[pl.ds(base, NODES_PT)])
    plsc.subcore_barrier()

    pltpu.async_copy(idxw_hbm.at[s, 0], icn0, sem_i0)
    pltpu.async_copy(idxw_hbm.at[s, 1], icn1, sem_i1)

    @pl.loop(0, NCHUNK // 2)
    def _(g):
        for u in range(2):
            cc = g * 2 + u
            buf, sem = icn[u], sem_i[u]
            pltpu.make_async_copy(idxw_hbm.at[s, cc], buf, sem).wait()

            @pl.loop(0, KE // 16)
            def _(i):
                slc = pl.ds(i * 16, 16)
                cnb[slc] = plsc.bitcast(buf[2, slc], jnp.float32)

            pltpu.sync_copy(cnb, deg_sh.at[buf.at[1]], add=True)
            pltpu.async_copy(
                idxw_hbm.at[s, lax.rem(cc + 2, NCHUNK)], buf, sem)

    pltpu.make_async_copy(idxw_hbm.at[s, 0], icn0, sem_i0).wait()
    pltpu.make_async_copy(idxw_hbm.at[s, 1], icn1, sem_i1).wait()

    plsc.subcore_barrier()

    pltpu.sync_copy(deg_sh.at[pl.ds(base, NODES_PT)], deg_t)

    @pl.loop(0, NODES_PT // 16)
    def _(i):
        deg_t[pl.ds(i * 16, 16)] = _rsqrt16(deg_t[pl.ds(i * 16, 16)])

    pltpu.sync_copy(deg_t, deg_sh.at[pl.ds(base, NODES_PT)])
    plsc.subcore_barrier()

    pltpu.sync_copy(deg_sh, dinv_t)

    pltpu.async_copy(idxw_hbm.at[s, 0], icn0, sem_i0)
    pltpu.async_copy(idxw_hbm.at[s, 1], icn1, sem_i1)

    @pl.loop(0, NCHUNK // 2)
    def _(g):
        for u in range(2):
            cc = g * 2 + u
            buf, sem = icn[u], sem_i[u]
            pltpu.make_async_copy(idxw_hbm.at[s, cc], buf, sem).wait()

            @pl.loop(0, KE // 16)
            def _(i):
                slc = pl.ds(i * 16, 16)
                dr = plsc.load_gather(dinv_t, [buf[0, slc]])
                dc = plsc.load_gather(dinv_t, [buf[1, slc]])
                w = plsc.bitcast(buf[2, slc], jnp.float32)
                buf[2, slc] = plsc.bitcast(w * dr * dc, jnp.int32)

            pltpu.sync_copy(buf, pk_hbm.at[s, cc])
            pltpu.async_copy(
                idxw_hbm.at[s, lax.rem(cc + 2, NCHUNK)], buf, sem)

    pltpu.make_async_copy(idxw_hbm.at[s, 0], icn0, sem_i0).wait()
    pltpu.make_async_copy(idxw_hbm.at[s, 1], icn1, sem_i1).wait()

    @pl.loop(0, NCH_NODE)
    def _(j):
        nbase = base + j * CHN
        pltpu.sync_copy(h_hbm.at[c, pl.ds(nbase, CHN)], na)
        pltpu.sync_copy(na, out_sh.at[pl.ds(nbase, CHN)])
        pltpu.sync_copy(zb, agg_sh.at[pl.ds(nbase, CHN)])

    plsc.subcore_barrier()

    def scale_rows(rb, ic):
        nrm = ic.at[2]

        @pl.loop(0, KE // 16)
        def _(i16):
            bvec = jnp.full((16,), i16 * 16, jnp.int32)
            for ii in range(16):
                sc16 = plsc.bitcast(
                    plsc.load_gather(nrm, [bvec + ii]), jnp.float32)
                r = i16 * 16 + ii
                for j in range(DH // 16):
                    slc = pl.ds(j * 16, 16)
                    rb[r, slc] = rb[r, slc] * sc16

    @pl.loop(0, K)
    def _(it):
        pltpu.sync_copy(pk_hbm.at[s, 0], icn0)
        pltpu.async_copy(pk_hbm.at[s, 1], icn1, sem_i1)
        @pl.loop(0, KE)
        def _(i):
            for j in range(DH // 16):
                rows1[i, pl.ds(j * 16, 16)] = zvec

        pltpu.async_copy(rows1, agg_sh.at[icn0.at[1]], sem_s1, add=True)
        pltpu.async_copy(out_sh.at[icn0.at[0]], rows0, sem_g0)

        @pl.loop(0, NGRP)
        def _(g):
            cbase = g * UGRP
            for u in range(UGRP):
                cc = cbase + u
                s2, s2n = u % 2, (u + 1) % 2
                s3, s3n, s3p = u % 3, (u + 1) % 3, (u + 2) % 3
                cn1 = lax.rem(cc + 1, NCHUNK)
                cn2 = lax.rem(cc + 2, NCHUNK)
                pltpu.make_async_copy(
                    pk_hbm.at[s, cn1], icn[s3n], sem_i[s3n]).wait()
                pltpu.make_async_copy(
                    out_sh.at[icn[s3].at[0]], rows[s2], sem_g[s2]).wait()
                pltpu.make_async_copy(
                    rows[s2n], agg_sh.at[icn[s3p].at[1]], sem_s[s2n]).wait()
                pltpu.async_copy(pk_hbm.at[s, cn2], icn[s3p], sem_i[s3p])
                pltpu.async_copy(
                    out_sh.at[icn[s3n].at[0]], rows[s2n], sem_g[s2n])
                scale_rows(rows[s2], icn[s3])
                pltpu.async_copy(
                    rows[s2], agg_sh.at[icn[s3].at[1]], sem_s[s2], add=True)

        pltpu.make_async_copy(
            rows[1], agg_sh.at[icn[2].at[1]], sem_s[1]).wait()
        pltpu.make_async_copy(out_sh.at[icn[0].at[0]], rows[0], sem_g[0]).wait()
        pltpu.make_async_copy(pk_hbm.at[s, 1], icn[1], sem_i[1]).wait()
        plsc.subcore_barrier()

        @pl.loop(0, NCH_NODE)
        def _(j):
            nbase = base + j * CHN
            pltpu.async_copy(agg_sh.at[pl.ds(nbase, CHN)], na, sem_g0)
            pltpu.async_copy(h_hbm.at[c, pl.ds(nbase, CHN)], nh, sem_g1)
            pltpu.make_async_copy(
                agg_sh.at[pl.ds(nbase, CHN)], na, sem_g0).wait()
            pltpu.make_async_copy(
                h_hbm.at[c, pl.ds(nbase, CHN)], nh, sem_g1).wait()
            pltpu.sync_copy(zb, agg_sh.at[pl.ds(nbase, CHN)])

            @pl.loop(0, CHN)
            def _(i):
                for jj in range(DH // 16):
                    slc = pl.ds(jj * 16, 16)
                    na[i, slc] = (1.0 - ALPHA) * na[i, slc] + ALPHA * nh[i, slc]

            pltpu.sync_copy(na, out_sh.at[pl.ds(nbase, CHN)])

        plsc.subcore_barrier()

    @pl.loop(0, NCH_NODE)
    def _(j):
        nbase = base + j * CHN
        pltpu.sync_copy(out_sh.at[pl.ds(nbase, CHN)], na)
        pltpu.sync_copy(na, out_hbm.at[c, pl.ds(nbase, CHN)])


def kernel(x, edge_index, edge_weight, W1, b1, W2, b2):
    x_pad = jnp.pad(x, ((0, NPAD - N), (0, 0)))
    h2 = _mlp(x_pad, W1, b1.reshape(1, D), W2, b2.reshape(1, D))

    npad_e = E_ALL - E - NPAD
    loop_idx = jnp.arange(NPAD, dtype=jnp.int32)
    zpad = jnp.zeros((npad_e,), jnp.int32)
    rows_all = jnp.concatenate([edge_index[0], loop_idx, zpad])
    cols_all = jnp.concatenate([edge_index[1], loop_idx, zpad])
    w_all = jnp.concatenate(
        [edge_weight, jnp.ones((NPAD,), jnp.float32),
         jnp.zeros((npad_e,), jnp.float32)])
    wbits = lax.bitcast_convert_type(w_all, jnp.int32)
    idxw = jnp.stack([rows_all.reshape(NS, NCHUNK, KE),
                      cols_all.reshape(NS, NCHUNK, KE),
                      wbits.reshape(NS, NCHUNK, KE)], axis=2)

    out2, _ = _appnp_sc(h2, idxw)
    return jnp.concatenate([out2[0], out2[1]], axis=1)[:N]
